# baseline (speedup 1.0000x reference)
import functools

import jax
import jax.numpy as jnp
from jax import lax
from jax.experimental import pallas as pl
from jax.experimental.pallas import tpu as pltpu
from jax.experimental.pallas import tpu_sc as plsc

D = 64
L = 16
CHUNK = 128


@functools.lru_cache(maxsize=None)
def _make_sc_kernel(B):
    info = plsc.get_sparse_core_info()
    NC, NS = info.num_cores, info.num_subcores
    NW = NC * NS
    bpw = B // NW
    nchunk = bpw // CHUNK
    assert B % (NW * CHUNK) == 0

    mesh = plsc.VectorSubcoreMesh(core_axis_name="c", subcore_axis_name="s")

    @functools.partial(
        pl.kernel,
        mesh=mesh,
        compiler_params=pltpu.CompilerParams(
            needs_layout_passes=False, use_tc_tiling_on_sc=False),
        out_type=jax.ShapeDtypeStruct((B,), jnp.float32),
        scratch_types=[
            pltpu.VMEM((nchunk, CHUNK), jnp.int32),
            pltpu.VMEM((nchunk, CHUNK), jnp.int32),
            pltpu.VMEM((bpw, D), jnp.float32),
            pltpu.VMEM((bpw, D), jnp.float32),
            pltpu.VMEM((bpw,), jnp.float32),
            pltpu.VMEM((L,), jnp.float32),
            pltpu.VMEM((L,), jnp.float32),
            pltpu.SemaphoreType.DMA,
        ],
    )
    def sc_kernel(idx_t_hbm, idx_c_hbm, table_hbm, w_hbm, b_hbm, out_hbm,
                  idx_t_v, idx_c_v, rows_t_v, rows_c_v, out_v, w_v, b_v, sem):
        wid = lax.axis_index("s") * NC + lax.axis_index("c")
        base = wid * bpw

        pltpu.sync_copy(idx_t_hbm.at[wid], idx_t_v)
        pltpu.sync_copy(idx_c_hbm.at[wid], idx_c_v)
        pltpu.sync_copy(w_hbm, w_v)
        pltpu.sync_copy(b_hbm, b_v)

        copies = []
        for j in range(nchunk):
            copies.append(pltpu.async_copy(
                table_hbm.at[idx_t_v.at[j]],
                rows_t_v.at[pl.ds(j * CHUNK, CHUNK)], sem))
            copies.append(pltpu.async_copy(
                table_hbm.at[idx_c_v.at[j]],
                rows_c_v.at[pl.ds(j * CHUNK, CHUNK)], sem))
        for cp in copies:
            cp.wait()

        wv = w_v[...]
        bv = b_v[...]

        def group_body(g, carry):
            rows = jnp.full((L,), g * L, jnp.int32) + lax.iota(jnp.int32, L)

            def col_body(d, acc):
                cidx = jnp.full((L,), d, jnp.int32)
                vt = plsc.load_gather(rows_t_v, [rows, cidx])
                vc = plsc.load_gather(rows_c_v, [rows, cidx])
                return acc + vt * vc

            acc = lax.fori_loop(0, D, col_body, jnp.zeros((L,), jnp.float32))
            z = acc * wv + bv
            out_v[pl.ds(g * L, L)] = 1.0 / (1.0 + jnp.exp(-z))
            return carry

        lax.fori_loop(0, bpw // L, group_body, 0)
        pltpu.sync_copy(out_v, out_hbm.at[pl.ds(base, bpw)])

    return sc_kernel, NW, nchunk


def kernel(input_target, input_context, embedding_table, dense_w, dense_b):
    B = input_target.shape[0]
    sc_kernel, NW, nchunk = _make_sc_kernel(B)
    idx_t = input_target.reshape(NW, nchunk, CHUNK).astype(jnp.int32)
    idx_c = input_context.reshape(NW, nchunk, CHUNK).astype(jnp.int32)
    w_arr = jnp.full((L,), dense_w[0, 0], jnp.float32)
    b_arr = jnp.full((L,), dense_b[0], jnp.float32)
    out = sc_kernel(idx_t, idx_c, embedding_table, w_arr, b_arr)
    return out.reshape(B, 1)


# speedup vs baseline: 1.6097x; 1.6097x over previous
import functools

import jax
import jax.numpy as jnp
from jax import lax
from jax.experimental import pallas as pl
from jax.experimental.pallas import tpu as pltpu
from jax.experimental.pallas import tpu_sc as plsc

D = 64
L = 16
PASSES = 2


@functools.lru_cache(maxsize=None)
def _make_sc_kernel(B):
    info = plsc.get_sparse_core_info()
    NC, NS = info.num_cores, info.num_subcores
    NW = NC * NS
    bpw = B // NW
    hrows = bpw // PASSES
    assert B % (NW * PASSES * L) == 0

    mesh = plsc.VectorSubcoreMesh(core_axis_name="c", subcore_axis_name="s")

    @functools.partial(
        pl.kernel,
        mesh=mesh,
        compiler_params=pltpu.CompilerParams(needs_layout_passes=False),
        out_type=jax.ShapeDtypeStruct((B,), jnp.float32),
        scratch_types=[
            pltpu.VMEM((bpw,), jnp.int32),
            pltpu.VMEM((bpw,), jnp.int32),
            pltpu.VMEM((hrows, D), jnp.float32),
            pltpu.VMEM((hrows, D), jnp.float32),
            pltpu.VMEM((bpw,), jnp.float32),
            pltpu.VMEM((L,), jnp.float32),
            pltpu.VMEM((L,), jnp.float32),
            pltpu.SemaphoreType.DMA,
        ],
    )
    def sc_kernel(idx_t_hbm, idx_c_hbm, table_hbm, w_hbm, b_hbm, out_hbm,
                  idx_t_v, idx_c_v, rows_t_v, rows_c_v, out_v, w_v, b_v, sem):
        wid = lax.axis_index("s") * NC + lax.axis_index("c")
        base = wid * bpw

        pltpu.sync_copy(idx_t_hbm.at[wid], idx_t_v)
        pltpu.sync_copy(idx_c_hbm.at[wid], idx_c_v)
        pltpu.sync_copy(w_hbm, w_v)
        pltpu.sync_copy(b_hbm, b_v)

        wv = w_v[...]
        bv = b_v[...]
        lane_iota = lax.iota(jnp.int32, L)

        for p in range(PASSES):
            poff = p * hrows

            def fire_body(g, carry):
                vt = idx_t_v[pl.ds(poff + g * L, L)]
                vc = idx_c_v[pl.ds(poff + g * L, L)]
                for l in range(L):
                    pltpu.async_copy(table_hbm.at[pl.ds(vt[l], 1), :],
                                     rows_t_v.at[pl.ds(g * L + l, 1), :], sem)
                    pltpu.async_copy(table_hbm.at[pl.ds(vc[l], 1), :],
                                     rows_c_v.at[pl.ds(g * L + l, 1), :], sem)
                return carry

            lax.fori_loop(0, hrows // L, fire_body, 0)

            def drain_body(i, carry):
                pltpu.make_async_copy(
                    table_hbm.at[pl.ds(0, 1), :], rows_t_v.at[pl.ds(0, 1), :],
                    sem).wait()
                return carry

            lax.fori_loop(0, 2 * hrows, drain_body, 0)

            def group_body(g, carry):
                rows = jnp.full((L,), g * L, jnp.int32) + lane_iota

                def col_body(d, acc):
                    cols = jnp.full((L,), d, jnp.int32)
                    vt = plsc.load_gather(rows_t_v, [rows, cols])
                    vc = plsc.load_gather(rows_c_v, [rows, cols])
                    return acc + vt * vc

                acc = lax.fori_loop(0, D, col_body,
                                    jnp.zeros((L,), jnp.float32))
                z = acc * wv + bv
                out_v[pl.ds(poff + g * L, L)] = 1.0 / (1.0 + jnp.exp(-z))
                return carry

            lax.fori_loop(0, hrows // L, group_body, 0)

        pltpu.sync_copy(out_v, out_hbm.at[pl.ds(base, bpw)])

    return sc_kernel, NW


def kernel(input_target, input_context, embedding_table, dense_w, dense_b):
    B = input_target.shape[0]
    sc_kernel, NW = _make_sc_kernel(B)
    idx_t = input_target.reshape(NW, B // NW).astype(jnp.int32)
    idx_c = input_context.reshape(NW, B // NW).astype(jnp.int32)
    w_arr = jnp.full((L,), dense_w[0, 0], jnp.float32)
    b_arr = jnp.full((L,), dense_b[0], jnp.float32)
    out = sc_kernel(idx_t, idx_c, embedding_table, w_arr, b_arr)
    return out.reshape(B, 1)
